# Initial kernel scaffold; baseline (speedup 1.0000x reference)
import jax
import jax.numpy as jnp
from jax.experimental import pallas as pl


def kernel(x, batch, W):
    raise NotImplementedError("write your pallas kernel here")


# speedup vs baseline: 2.3486x; 2.3486x over previous
import functools

import jax
import jax.numpy as jnp
from jax import lax
from jax.experimental import pallas as pl
from jax.experimental.pallas import tpu as pltpu
from jax.experimental.pallas import tpu_sc as plsc

N = 100000
D = 128
G = 256
NC = 2
NS = 16
NW = NC * NS
L = 16

CHUNK = 400
NUM_CHUNKS = N // CHUNK
KMAX = -(-NUM_CHUNKS // NW)


def _sc_segment_max_partial(x, batch):
    mesh = plsc.VectorSubcoreMesh(core_axis_name="c", subcore_axis_name="s",
                                  num_cores=NC, num_subcores=NS)

    @functools.partial(
        pl.kernel,
        out_type=jax.ShapeDtypeStruct((NW, G, D), jnp.float32),
        mesh=mesh,
        scratch_types=[
            pltpu.VMEM((G, D), jnp.float32),
            pltpu.VMEM((CHUNK, D), jnp.float32),
            pltpu.VMEM((CHUNK,), jnp.int32),
        ],
    )
    def seg_max(x_hbm, b_hbm, out_hbm, acc_v, xb_v, ids_v):
        wid = lax.axis_index("c") * NS + lax.axis_index("s")

        neg_inf = jnp.full((L,), -jnp.inf, dtype=jnp.float32)

        @pl.loop(0, G)
        def _init(g):
            for j in range(D // L):
                acc_v[g, pl.ds(j * L, L)] = neg_inf

        @pl.loop(0, KMAX)
        def _chunk_loop(k):
            chunk = wid + k * NW

            @pl.when(chunk < NUM_CHUNKS)
            def _():
                base = chunk * CHUNK
                pltpu.sync_copy(b_hbm.at[pl.ds(base, CHUNK)], ids_v)
                pltpu.sync_copy(x_hbm.at[pl.ds(base, CHUNK)], xb_v)

                @pl.loop(0, CHUNK // L)
                def _grp(gi):
                    row0 = gi * L
                    idv = ids_v[pl.ds(row0, L)]
                    for t in range(L):
                        b = idv[t]
                        for j in range(D // L):
                            sl = pl.ds(j * L, L)
                            acc_v[b, sl] = jnp.maximum(acc_v[b, sl],
                                                       xb_v[row0 + t, sl])

        pltpu.sync_copy(acc_v, out_hbm.at[wid])

    return seg_max(x, batch)


def _tc_combine_matmul(partial, W):
    def body(p_ref, w_ref, out_ref):
        hg = jnp.max(p_ref[...], axis=0)
        out_ref[...] = lax.dot_general(
            hg, w_ref[...], (((1,), (1,)), ((), ())),
            preferred_element_type=jnp.float32)

    return pl.pallas_call(
        body,
        out_shape=jax.ShapeDtypeStruct((G, W.shape[0]), jnp.float32),
    )(partial, W)


def kernel(x, batch, W):
    partial = _sc_segment_max_partial(x, batch.astype(jnp.int32))
    logits = _tc_combine_matmul(partial, W)
    return (logits, logits)


# speedup vs baseline: 4.3324x; 1.8446x over previous
import functools

import jax
import jax.numpy as jnp
from jax import lax
from jax.experimental import pallas as pl
from jax.experimental.pallas import tpu as pltpu
from jax.experimental.pallas import tpu_sc as plsc

N = 100000
D = 128
G = 256
NC = 2
NS = 16
NW = NC * NS
L = 16

CHUNK = 400
NUM_CHUNKS = N // CHUNK
KMAX = -(-NUM_CHUNKS // NW)


def _sc_segment_max_partial(x, batch):
    mesh = plsc.VectorSubcoreMesh(core_axis_name="c", subcore_axis_name="s",
                                  num_cores=NC, num_subcores=NS)

    @functools.partial(
        pl.kernel,
        out_type=jax.ShapeDtypeStruct((NW, G, D), jnp.float32),
        mesh=mesh,
        scratch_types=[
            pltpu.VMEM((G + 1, D), jnp.float32),
            pltpu.VMEM((CHUNK, D), jnp.float32),
            pltpu.VMEM((CHUNK,), jnp.int32),
        ],
    )
    def seg_max(x_hbm, b_hbm, out_hbm, acc_v, xb_v, ids_v):
        wid = lax.axis_index("c") * NS + lax.axis_index("s")
        NV = D // L

        neg_inf = jnp.full((L,), -jnp.inf, dtype=jnp.float32)

        @pl.loop(0, G + 1)
        def _init(g):
            for j in range(NV):
                acc_v[g, pl.ds(j * L, L)] = neg_inf

        def rmw(row, vals):
            for j in range(NV):
                sl = pl.ds(j * L, L)
                acc_v[row, sl] = jnp.maximum(acc_v[row, sl], vals[j])

        def grp_body(gi, carry):
            cur, accs = carry
            row0 = gi * L
            idv = ids_v[pl.ds(row0, L)]
            first = idv[0]
            last = idv[L - 1]
            pred_fast = (first == cur) & (last == cur)

            @pl.when(jnp.logical_not(pred_fast))
            def _slow():
                rmw(cur, accs)
                for t in range(L):
                    bt = idv[t]
                    rmw(bt, tuple(xb_v[row0 + t, pl.ds(j * L, L)]
                                  for j in range(NV)))

            new_accs = accs
            for t in range(L):
                new_accs = tuple(
                    jnp.maximum(a, xb_v[row0 + t, pl.ds(j * L, L)])
                    for j, a in enumerate(new_accs))
            accs = tuple(jnp.where(pred_fast, a, neg_inf) for a in new_accs)
            cur = jnp.where(pred_fast, cur, last)
            return (cur, accs)

        def chunk_body(k, carry):
            chunk = wid + k * NW
            base = chunk * CHUNK
            pltpu.sync_copy(b_hbm.at[pl.ds(base, CHUNK)], ids_v)
            pltpu.sync_copy(x_hbm.at[pl.ds(base, CHUNK)], xb_v)
            return lax.fori_loop(0, CHUNK // L, grp_body, carry)

        nk = (NUM_CHUNKS - 1 - wid) // NW + 1
        cur0 = jnp.int32(G)
        accs0 = tuple(neg_inf for _ in range(NV))
        cur, accs = lax.fori_loop(0, nk, chunk_body, (cur0, accs0))
        rmw(cur, accs)

        pltpu.sync_copy(acc_v.at[pl.ds(0, G)], out_hbm.at[wid])

    return seg_max(x, batch)


def _tc_combine_matmul(partial, W):
    def body(p_ref, w_ref, out_ref):
        hg = jnp.max(p_ref[...], axis=0)
        out_ref[...] = lax.dot_general(
            hg, w_ref[...], (((1,), (1,)), ((), ())),
            preferred_element_type=jnp.float32)

    return pl.pallas_call(
        body,
        out_shape=jax.ShapeDtypeStruct((G, W.shape[0]), jnp.float32),
    )(partial, W)


def kernel(x, batch, W):
    partial = _sc_segment_max_partial(x, batch.astype(jnp.int32))
    logits = _tc_combine_matmul(partial, W)
    return (logits, logits)


# speedup vs baseline: 5.7198x; 1.3202x over previous
import functools

import jax
import jax.numpy as jnp
from jax import lax
from jax.experimental import pallas as pl
from jax.experimental.pallas import tpu as pltpu
from jax.experimental.pallas import tpu_sc as plsc

N = 100000
D = 128
G = 256
NC = 2
NS = 16
NW = NC * NS
L = 16

CHUNK = 160
NUM_CHUNKS = N // CHUNK


def _sc_segment_max_partial(x, batch):
    mesh = plsc.VectorSubcoreMesh(core_axis_name="c", subcore_axis_name="s",
                                  num_cores=NC, num_subcores=NS)

    @functools.partial(
        pl.kernel,
        out_type=jax.ShapeDtypeStruct((NW, G, D), jnp.float32),
        mesh=mesh,
        scratch_types=[
            pltpu.VMEM((G + 1, D), jnp.float32),
            pltpu.VMEM((2, CHUNK, D), jnp.float32),
            pltpu.VMEM((2, 1, CHUNK), jnp.int32),
            pltpu.SemaphoreType.DMA((2,)),
            pltpu.SemaphoreType.DMA((2,)),
        ],
    )
    def seg_max(x_hbm, b_hbm, out_hbm, acc_v, xb_v, ids_v, sem_x, sem_i):
        wid = lax.axis_index("c") * NS + lax.axis_index("s")
        NV = D // L

        neg_inf = jnp.full((L,), -jnp.inf, dtype=jnp.float32)

        @pl.loop(0, G + 1)
        def _init(g):
            for j in range(NV):
                acc_v[g, pl.ds(j * L, L)] = neg_inf

        def rmw(row, vals):
            for j in range(NV):
                sl = pl.ds(j * L, L)
                acc_v[row, sl] = jnp.maximum(acc_v[row, sl], vals[j])

        def start_dma(k, buf):
            chunk = wid + k * NW
            pltpu.async_copy(b_hbm.at[chunk], ids_v.at[buf], sem_i.at[buf])
            pltpu.async_copy(x_hbm.at[pl.ds(chunk * CHUNK, CHUNK)],
                             xb_v.at[buf], sem_x.at[buf])

        def wait_dma(k, buf):
            chunk = wid + k * NW
            pltpu.make_async_copy(b_hbm.at[chunk], ids_v.at[buf],
                                  sem_i.at[buf]).wait()
            pltpu.make_async_copy(x_hbm.at[pl.ds(chunk * CHUNK, CHUNK)],
                                  xb_v.at[buf], sem_x.at[buf]).wait()

        def grp_body(gi, carry, buf):
            cur, accs = carry
            row0 = gi * L
            idv = ids_v[buf, 0, pl.ds(row0, L)]
            first = idv[0]
            last = idv[L - 1]
            pred_fast = (first == cur) & (last == cur)

            @pl.when(jnp.logical_not(pred_fast))
            def _slow():
                rmw(cur, accs)
                for t in range(L):
                    bt = idv[t]
                    rmw(bt, tuple(xb_v[buf, row0 + t, pl.ds(j * L, L)]
                                  for j in range(NV)))

            new_accs = accs
            for t in range(L):
                new_accs = tuple(
                    jnp.maximum(a, xb_v[buf, row0 + t, pl.ds(j * L, L)])
                    for j, a in enumerate(new_accs))
            accs = tuple(jnp.where(pred_fast, a, neg_inf) for a in new_accs)
            cur = jnp.where(pred_fast, cur, last)
            return (cur, accs)

        nk = (NUM_CHUNKS - 1 - wid) // NW + 1

        def chunk_body(k, carry):
            buf = lax.rem(k, 2)

            @pl.when(k + 1 < nk)
            def _prefetch():
                start_dma(k + 1, 1 - buf)

            wait_dma(k, buf)
            return lax.fori_loop(0, CHUNK // L,
                                 lambda gi, c: grp_body(gi, c, buf), carry)

        start_dma(0, 0)
        cur0 = jnp.int32(G)
        accs0 = tuple(neg_inf for _ in range(NV))
        cur, accs = lax.fori_loop(0, nk, chunk_body, (cur0, accs0))
        rmw(cur, accs)

        pltpu.sync_copy(acc_v.at[pl.ds(0, G)], out_hbm.at[wid])

    return seg_max(x, batch.reshape(NUM_CHUNKS, 1, CHUNK))


def _tc_combine_matmul(partial, W):
    def body(p_ref, w_ref, out_ref):
        hg = jnp.max(p_ref[...], axis=0)
        out_ref[...] = lax.dot_general(
            hg, w_ref[...], (((1,), (1,)), ((), ())),
            preferred_element_type=jnp.float32)

    return pl.pallas_call(
        body,
        out_shape=jax.ShapeDtypeStruct((G, W.shape[0]), jnp.float32),
    )(partial, W)


def kernel(x, batch, W):
    partial = _sc_segment_max_partial(x, batch.astype(jnp.int32))
    logits = _tc_combine_matmul(partial, W)
    return (logits, logits)


# speedup vs baseline: 6.0219x; 1.0528x over previous
import functools

import jax
import jax.numpy as jnp
from jax import lax
from jax.experimental import pallas as pl
from jax.experimental.pallas import tpu as pltpu
from jax.experimental.pallas import tpu_sc as plsc

N = 100000
D = 128
G = 256
NC = 2
NS = 16
NW = NC * NS
L = 16

CHUNK = 160
NUM_CHUNKS = N // CHUNK
CPW = -(-NUM_CHUNKS // NW)
GPC = CHUNK // L


def _sc_segment_max_partial(x, batch):
    mesh = plsc.VectorSubcoreMesh(core_axis_name="c", subcore_axis_name="s",
                                  num_cores=NC, num_subcores=NS)

    @functools.partial(
        pl.kernel,
        out_type=jax.ShapeDtypeStruct((NW, G, D), jnp.float32),
        mesh=mesh,
        scratch_types=[
            pltpu.VMEM((G + 1, D), jnp.float32),
            pltpu.VMEM((2, CHUNK, D), jnp.float32),
            pltpu.VMEM((2 * CHUNK,), jnp.int32),
            pltpu.SemaphoreType.DMA((2,)),
            pltpu.SemaphoreType.DMA((2,)),
        ],
    )
    def seg_max(x_hbm, b_hbm, out_hbm, acc_v, xb_v, ids_v, sem_x, sem_i):
        wid = lax.axis_index("c") * NS + lax.axis_index("s")
        NV = D // L

        cw = (wid * (NUM_CHUNKS - CPW)) // (NW - 1)

        neg_inf = jnp.full((L,), -jnp.inf, dtype=jnp.float32)

        @pl.loop(0, G + 1)
        def _init(g):
            for j in range(NV):
                acc_v[g, pl.ds(j * L, L)] = neg_inf

        def rmw(row, vals):
            for j in range(NV):
                sl = pl.ds(j * L, L)
                acc_v[row, sl] = jnp.maximum(acc_v[row, sl], vals[j])

        def start_dma(k, buf):
            base = (cw + k) * CHUNK
            pltpu.async_copy(b_hbm.at[pl.ds(base, CHUNK)],
                             ids_v.at[pl.ds(buf * CHUNK, CHUNK)],
                             sem_i.at[buf])
            pltpu.async_copy(x_hbm.at[pl.ds(base, CHUNK)],
                             xb_v.at[buf], sem_x.at[buf])

        def wait_dma(k, buf):
            base = (cw + k) * CHUNK
            pltpu.make_async_copy(b_hbm.at[pl.ds(base, CHUNK)],
                                  ids_v.at[pl.ds(buf * CHUNK, CHUNK)],
                                  sem_i.at[buf]).wait()
            pltpu.make_async_copy(x_hbm.at[pl.ds(base, CHUNK)],
                                  xb_v.at[buf], sem_x.at[buf]).wait()

        def make_grp_body(buf):
            def grp_body(gi, carry):
                cur, accs = carry
                row0 = gi * L
                idv = ids_v[pl.ds(buf * CHUNK + row0, L)]
                first = idv[0]
                last = idv[L - 1]
                pred_fast = (first == cur) & (last == cur)

                @pl.when(jnp.logical_not(pred_fast))
                def _slow():
                    rmw(cur, accs)
                    for t in range(L):
                        bt = idv[t]
                        rmw(bt, tuple(xb_v[buf, row0 + t, pl.ds(j * L, L)]
                                      for j in range(NV)))

                new_accs = accs
                for t in range(L):
                    new_accs = tuple(
                        jnp.maximum(a, xb_v[buf, row0 + t, pl.ds(j * L, L)])
                        for j, a in enumerate(new_accs))
                accs = tuple(jnp.where(pred_fast, a, neg_inf)
                             for a in new_accs)
                cur = jnp.where(pred_fast, cur, last)
                return (cur, accs)
            return grp_body

        grp0 = make_grp_body(0)
        grp1 = make_grp_body(1)

        def pair_body(p, carry):
            k0 = 2 * p
            start_dma(k0 + 1, 1)
            wait_dma(k0, 0)
            carry = lax.fori_loop(0, GPC, grp0, carry)

            @pl.when(k0 + 2 < CPW)
            def _():
                start_dma(k0 + 2, 0)

            wait_dma(k0 + 1, 1)
            carry = lax.fori_loop(0, GPC, grp1, carry)
            return carry

        start_dma(0, 0)
        cur0 = jnp.int32(G)
        accs0 = tuple(neg_inf for _ in range(NV))
        cur, accs = lax.fori_loop(0, CPW // 2, pair_body, (cur0, accs0))
        rmw(cur, accs)

        pltpu.sync_copy(acc_v.at[pl.ds(0, G)], out_hbm.at[wid])

    return seg_max(x, batch)


def _tc_combine_matmul(partial, W):
    def body(p_ref, w_ref, out_ref):
        hg = jnp.max(p_ref[...], axis=0)
        out_ref[...] = lax.dot_general(
            hg, w_ref[...], (((1,), (1,)), ((), ())),
            preferred_element_type=jnp.float32)

    return pl.pallas_call(
        body,
        out_shape=jax.ShapeDtypeStruct((G, W.shape[0]), jnp.float32),
    )(partial, W)


def kernel(x, batch, W):
    partial = _sc_segment_max_partial(x, batch.astype(jnp.int32))
    logits = _tc_combine_matmul(partial, W)
    return (logits, logits)


# speedup vs baseline: 6.3951x; 1.0620x over previous
import functools

import jax
import jax.numpy as jnp
from jax import lax
from jax.experimental import pallas as pl
from jax.experimental.pallas import tpu as pltpu
from jax.experimental.pallas import tpu_sc as plsc

N = 100000
D = 128
G = 256
NC = 2
NS = 16
NW = NC * NS
L = 16

CHUNK = 160
NUM_CHUNKS = N // CHUNK
CPW = -(-NUM_CHUNKS // NW)
GPC = CHUNK // L


def _sc_segment_max_partial(x, batch):
    mesh = plsc.VectorSubcoreMesh(core_axis_name="c", subcore_axis_name="s",
                                  num_cores=NC, num_subcores=NS)

    @functools.partial(
        pl.kernel,
        out_type=jax.ShapeDtypeStruct((NW, G, D), jnp.float32),
        mesh=mesh,
        scratch_types=[
            pltpu.VMEM((G, D), jnp.float32),
            pltpu.VMEM((2, CHUNK, D), jnp.float32),
            pltpu.VMEM((2 * CHUNK,), jnp.int32),
            pltpu.SemaphoreType.DMA((2,)),
            pltpu.SemaphoreType.DMA((2,)),
        ],
    )
    def seg_max(x_hbm, b_hbm, out_hbm, acc_v, xb_v, ids_v, sem_x, sem_i):
        wid = lax.axis_index("c") * NS + lax.axis_index("s")
        NV = D // L

        cw = (wid * (NUM_CHUNKS - CPW)) // (NW - 1)

        neg_inf = jnp.full((L,), -jnp.inf, dtype=jnp.float32)

        @pl.loop(0, G)
        def _init(g):
            for j in range(NV):
                acc_v[g, pl.ds(j * L, L)] = neg_inf

        def rmw(row, vals):
            for j in range(NV):
                sl = pl.ds(j * L, L)
                acc_v[row, sl] = jnp.maximum(acc_v[row, sl], vals[j])

        def start_dma(k, buf):
            base = (cw + k) * CHUNK
            pltpu.async_copy(b_hbm.at[pl.ds(base, CHUNK)],
                             ids_v.at[pl.ds(buf * CHUNK, CHUNK)],
                             sem_i.at[buf])
            pltpu.async_copy(x_hbm.at[pl.ds(base, CHUNK)],
                             xb_v.at[buf], sem_x.at[buf])

        def wait_dma(k, buf):
            base = (cw + k) * CHUNK
            pltpu.make_async_copy(b_hbm.at[pl.ds(base, CHUNK)],
                                  ids_v.at[pl.ds(buf * CHUNK, CHUNK)],
                                  sem_i.at[buf]).wait()
            pltpu.make_async_copy(x_hbm.at[pl.ds(base, CHUNK)],
                                  xb_v.at[buf], sem_x.at[buf]).wait()

        def process_chunk(buf):
            @pl.loop(0, GPC)
            def _grp(gi):
                row0 = gi * L
                idv = ids_v[pl.ds(buf * CHUNK + row0, L)]
                first = idv[0]
                last = idv[L - 1]

                @pl.when(first == last)
                def _fast():
                    accs = tuple(xb_v[buf, row0, pl.ds(j * L, L)]
                                 for j in range(NV))
                    for t in range(1, L):
                        accs = tuple(
                            jnp.maximum(a, xb_v[buf, row0 + t,
                                                pl.ds(j * L, L)])
                            for j, a in enumerate(accs))
                    rmw(first, accs)

                @pl.when(first != last)
                def _slow():
                    for t in range(L):
                        bt = idv[t]
                        rmw(bt, tuple(xb_v[buf, row0 + t, pl.ds(j * L, L)]
                                      for j in range(NV)))

        start_dma(0, 0)

        @pl.loop(0, CPW // 2)
        def _pair(p):
            k0 = 2 * p
            start_dma(k0 + 1, 1)
            wait_dma(k0, 0)
            process_chunk(0)

            @pl.when(k0 + 2 < CPW)
            def _():
                start_dma(k0 + 2, 0)

            wait_dma(k0 + 1, 1)
            process_chunk(1)

        pltpu.sync_copy(acc_v.at[pl.ds(0, G)], out_hbm.at[wid])

    return seg_max(x, batch)


def _tc_combine_matmul(partial, W):
    def body(p_ref, w_ref, out_ref):
        hg = jnp.max(p_ref[...], axis=0)
        out_ref[...] = lax.dot_general(
            hg, w_ref[...], (((1,), (1,)), ((), ())),
            preferred_element_type=jnp.float32)

    return pl.pallas_call(
        body,
        out_shape=jax.ShapeDtypeStruct((G, W.shape[0]), jnp.float32),
    )(partial, W)


def kernel(x, batch, W):
    partial = _sc_segment_max_partial(x, batch.astype(jnp.int32))
    logits = _tc_combine_matmul(partial, W)
    return (logits, logits)
